# baseline (speedup 1.0000x reference)
import functools
import math

import jax
import jax.numpy as jnp
from jax import lax
from jax.experimental import pallas as pl
from jax.experimental.pallas import tpu as pltpu
from jax.experimental.pallas import tpu_sc as plsc

VOCAB = 1000000
EMB = 64
B = 1024
L = 200
N = B * L

NC = 2
NS = 16
NW = NC * NS
NIDX = N // NW
CHUNK = 80
NCHUNK = NIDX // CHUNK
NBUF = 4
LOOKAHEAD = 2
NGROUP = NCHUNK // NBUF
SCALE = math.sqrt(EMB)


def _body(tok_hbm, table_hbm, out_hbm, idx_v, par_v, gbufs, obufs, gsems,
          ssems):
    wid = lax.axis_index("s") * NC + lax.axis_index("c")
    t0 = wid * NIDX

    pltpu.sync_copy(tok_hbm.at[pl.ds(t0, NIDX)], idx_v)

    @pl.loop(0, NIDX // 16, unroll=8)
    def _(i):
        sl = pl.ds(i * 16, 16)
        v = idx_v[sl]
        par_v[sl] = lax.bitwise_and(v, 1)
        idx_v[sl] = lax.shift_right_logical(v, 1)

    def src(j):
        return table_hbm.at[idx_v.at[pl.ds(j * CHUNK, CHUNK)]]

    def dst(j):
        return out_hbm.at[pl.ds((t0 + j * CHUNK) * EMB, CHUNK * EMB)]

    def gather_start(j, b):
        pltpu.async_copy(src(j), gbufs[b], gsems[b])

    def gather_wait(j, b):
        pltpu.make_async_copy(src(j), gbufs[b], gsems[b]).wait()

    def store_start(j, b):
        pltpu.async_copy(obufs[b], dst(j), ssems[b])

    def store_wait(j, b):
        pltpu.make_async_copy(obufs[b], dst(j), ssems[b]).wait()

    def select_scale(j, b):
        gbuf = gbufs[b]
        obuf = obufs[b]

        @pl.loop(0, CHUNK // 16)
        def _(g):
            par16 = par_v[pl.ds(j * CHUNK + g * 16, 16)] * EMB
            for rr in range(16):
                r = g * 16 + rr
                off = par16[rr]
                for c in range(EMB // 16):
                    obuf[pl.ds(r * EMB + c * 16, 16)] = (
                        gbuf[r, pl.ds(off + c * 16, 16)] * SCALE)

    for j in range(LOOKAHEAD):
        gather_start(j, j)

    @pl.loop(0, NGROUP)
    def _(g):
        for b in range(NBUF):
            j = g * NBUF + b
            f = j + LOOKAHEAD
            fb = (b + LOOKAHEAD) % NBUF

            @pl.when(f < NCHUNK)
            def _():
                gather_start(f, fb)

            gather_wait(j, b)

            @pl.when(j >= NBUF)
            def _():
                store_wait(j - NBUF, b)

            select_scale(j, b)
            store_start(j, b)

    for j in range(NCHUNK - NBUF, NCHUNK):
        store_wait(j, j % NBUF)


@functools.partial(jax.jit, static_argnames=())
def kernel(tokens, table):
    sc_gather = pl.kernel(
        _body,
        out_type=jax.ShapeDtypeStruct((N * EMB,), jnp.float32),
        mesh=plsc.VectorSubcoreMesh(core_axis_name="c", subcore_axis_name="s"),
        scratch_types=dict(
            idx_v=pltpu.VMEM((NIDX,), jnp.int32),
            par_v=pltpu.VMEM((NIDX,), jnp.int32),
            gbufs=[pltpu.VMEM((CHUNK, 2 * EMB), jnp.float32)
                   for _ in range(NBUF)],
            obufs=[pltpu.VMEM((CHUNK * EMB,), jnp.float32)
                   for _ in range(NBUF)],
            gsems=[pltpu.SemaphoreType.DMA for _ in range(NBUF)],
            ssems=[pltpu.SemaphoreType.DMA for _ in range(NBUF)],
        ),
        compiler_params=pltpu.CompilerParams(use_tc_tiling_on_sc=False),
    )
    out = sc_gather(tokens.astype(jnp.int32).reshape(-1),
                    table.reshape(VOCAB // 2, 2 * EMB))
    return out.reshape(B, L, EMB)


# speedup vs baseline: 1.1069x; 1.1069x over previous
import functools
import math

import jax
import jax.numpy as jnp
from jax import lax
from jax.experimental import pallas as pl
from jax.experimental.pallas import tpu as pltpu
from jax.experimental.pallas import tpu_sc as plsc

VOCAB = 1000000
EMB = 64
B = 1024
L = 200
N = B * L

NC = 2
NS = 16
NW = NC * NS
NIDX = N // NW
CHUNK = 80
NCHUNK = NIDX // CHUNK
NBUF = 4
LOOKAHEAD = 2
NGROUP = NCHUNK // NBUF
SCALE = math.sqrt(EMB)


def _body(tok_hbm, table_hbm, out_hbm, idx_v, gbufs, obufs, gsems, ssems):
    wid = lax.axis_index("s") * NC + lax.axis_index("c")
    t0 = wid * NIDX

    pltpu.sync_copy(tok_hbm.at[pl.ds(t0, NIDX)], idx_v)

    def src(j):
        return table_hbm.at[idx_v.at[pl.ds(j * CHUNK, CHUNK)]]

    def dst(j):
        return out_hbm.at[pl.ds((t0 + j * CHUNK) * EMB, CHUNK * EMB)]

    def gather_start(j, b):
        pltpu.async_copy(src(j), gbufs[b], gsems[b])

    def gather_wait(j, b):
        pltpu.make_async_copy(src(j), gbufs[b], gsems[b]).wait()

    def store_start(j, b):
        pltpu.async_copy(obufs[b], dst(j), ssems[b])

    def store_wait(j, b):
        pltpu.make_async_copy(obufs[b], dst(j), ssems[b]).wait()

    def scale(b):
        gbuf = gbufs[b]
        obuf = obufs[b]

        @pl.loop(0, CHUNK, unroll=4)
        def _(r):
            for c in range(EMB // 16):
                obuf[pl.ds(r * EMB + c * 16, 16)] = (
                    gbuf[r, pl.ds(c * 16, 16)] * SCALE)

    for j in range(LOOKAHEAD):
        gather_start(j, j)

    @pl.loop(0, NGROUP)
    def _(g):
        for b in range(NBUF):
            j = g * NBUF + b
            f = j + LOOKAHEAD
            fb = (b + LOOKAHEAD) % NBUF

            @pl.when(f < NCHUNK)
            def _():
                gather_start(f, fb)

            gather_wait(j, b)

            @pl.when(j >= NBUF)
            def _():
                store_wait(j - NBUF, b)

            scale(b)
            store_start(j, b)

    for j in range(NCHUNK - NBUF, NCHUNK):
        store_wait(j, j % NBUF)


@functools.partial(jax.jit, static_argnames=())
def kernel(tokens, table):
    sc_gather = pl.kernel(
        _body,
        out_type=jax.ShapeDtypeStruct((N * EMB,), jnp.float32),
        mesh=plsc.VectorSubcoreMesh(core_axis_name="c", subcore_axis_name="s"),
        scratch_types=dict(
            idx_v=pltpu.VMEM((NIDX,), jnp.int32),
            gbufs=[pltpu.VMEM((CHUNK, 2 * EMB), jnp.float32)
                   for _ in range(NBUF)],
            obufs=[pltpu.VMEM((CHUNK * EMB,), jnp.float32)
                   for _ in range(NBUF)],
            gsems=[pltpu.SemaphoreType.DMA for _ in range(NBUF)],
            ssems=[pltpu.SemaphoreType.DMA for _ in range(NBUF)],
        ),
        compiler_params=pltpu.CompilerParams(use_tc_tiling_on_sc=False),
    )
    tablep = jnp.pad(table, ((0, 0), (0, EMB)))
    out = sc_gather(tokens.astype(jnp.int32).reshape(-1), tablep)
    return out.reshape(B, L, EMB)


# speedup vs baseline: 1.1339x; 1.0244x over previous
import functools
import math

import jax
import jax.numpy as jnp
from jax import lax
from jax.experimental import pallas as pl
from jax.experimental.pallas import tpu as pltpu
from jax.experimental.pallas import tpu_sc as plsc

VOCAB = 1000000
EMB = 64
B = 1024
L = 200

NC = 2
NS = 16
NW = NC * NS
ROWS = B // NW
NCHUNK = ROWS
CHUNK = L
NBUF = 4
LOOKAHEAD = 2
NGROUP = NCHUNK // NBUF
SCALE = math.sqrt(EMB)


def _body(tok_hbm, table_hbm, out_hbm, idx_v, bufs, gsems, ssems):
    wid = lax.axis_index("s") * NC + lax.axis_index("c")
    row0 = wid * ROWS

    pltpu.sync_copy(tok_hbm.at[pl.ds(row0, ROWS)], idx_v)

    def src(j):
        return table_hbm.at[idx_v.at[j]]

    def dst(j):
        return out_hbm.at[row0 + j]

    def gather_start(j, b):
        pltpu.async_copy(src(j), bufs[b], gsems[b])

    def gather_wait(j, b):
        pltpu.make_async_copy(src(j), bufs[b], gsems[b]).wait()

    def store_start(j, b):
        pltpu.async_copy(bufs[b], dst(j), ssems[b])

    def store_wait(j, b):
        pltpu.make_async_copy(bufs[b], dst(j), ssems[b]).wait()

    def scale(b):
        buf = bufs[b]

        @pl.loop(0, CHUNK, unroll=4)
        def _(r):
            for c in range(EMB // 16):
                sl = (r, pl.ds(c * 16, 16))
                buf[sl] = buf[sl] * SCALE

    for j in range(LOOKAHEAD):
        gather_start(j, j)

    @pl.loop(0, NGROUP)
    def _(g):
        for b in range(NBUF):
            j = g * NBUF + b
            f = j + LOOKAHEAD
            fb = (b + LOOKAHEAD) % NBUF

            @pl.when(f < NCHUNK)
            def _():
                @pl.when(f >= NBUF)
                def _():
                    store_wait(f - NBUF, fb)

                gather_start(f, fb)

            gather_wait(j, b)
            scale(b)
            store_start(j, b)

    for j in range(NCHUNK - LOOKAHEAD, NCHUNK):
        store_wait(j, j % NBUF)


@functools.partial(jax.jit, static_argnames=())
def kernel(tokens, table):
    sc_gather = pl.kernel(
        _body,
        out_type=jax.ShapeDtypeStruct((B, L, EMB), jnp.float32),
        mesh=plsc.VectorSubcoreMesh(core_axis_name="c", subcore_axis_name="s"),
        scratch_types=dict(
            idx_v=pltpu.VMEM((ROWS, L), jnp.int32),
            bufs=[pltpu.VMEM((CHUNK, EMB), jnp.float32) for _ in range(NBUF)],
            gsems=[pltpu.SemaphoreType.DMA for _ in range(NBUF)],
            ssems=[pltpu.SemaphoreType.DMA for _ in range(NBUF)],
        ),
        compiler_params=pltpu.CompilerParams(use_tc_tiling_on_sc=False),
    )
    return sc_gather(tokens.astype(jnp.int32), table)
